# baseline (speedup 1.0000x reference)
import functools

import jax
import jax.numpy as jnp
from jax import lax
from jax.experimental import pallas as pl
from jax.experimental.pallas import tpu as pltpu
from jax.experimental.pallas import tpu_sc as plsc

T = 2048
H = 768
F = 1024
E = 16
BLK = 288
BT = 256
NBLK = -(-T * 2 // BLK) + E
NSLOTS = NBLK * BLK

NC, NS = 2, 16
NW = NC * NS


def _router_body(x_ref, rw_ref, rb_ref,
                 slot_ref, w_ref, eob_ref, act_ref,
                 scan_ref):
    x = x_ref[...]
    logits = lax.dot_general(
        x, rw_ref[...], (((1,), (1,)), ((), ())),
        preferred_element_type=jnp.float32) + rb_ref[...]

    ids = lax.broadcasted_iota(jnp.int32, (T, E), 1)
    m1 = jnp.max(logits, axis=1, keepdims=True)
    i1 = jnp.min(jnp.where(logits == m1, ids, E), axis=1, keepdims=True)
    masked = jnp.where(ids == i1, -jnp.inf, logits)
    m2 = jnp.max(masked, axis=1, keepdims=True)
    i2 = jnp.min(jnp.where(masked == m2, ids, E), axis=1, keepdims=True)

    e2 = jnp.exp(m2 - m1)
    p1 = 1.0 / (1.0 + e2)
    p2 = 1.0 - p1
    w_ref[0:T, :] = jnp.broadcast_to(p1, (T, E))
    w_ref[T:2 * T, :] = jnp.broadcast_to(p2, (T, E))

    h1 = (ids == i1).astype(jnp.float32)
    h2 = (ids == i2).astype(jnp.float32)
    hsum = h1 + h2

    rr = lax.broadcasted_iota(jnp.int32, (BT, BT), 0)
    cc = lax.broadcasted_iota(jnp.int32, (BT, BT), 1)
    tril = (rr > cc).astype(jnp.float32)
    carry = jnp.zeros((1, E), jnp.float32)
    for b in range(T // BT):
        hb = hsum[b * BT:(b + 1) * BT, :]
        scan_ref[b * BT:(b + 1) * BT, :] = carry + lax.dot_general(
            tril, hb, (((1,), (0,)), ((), ())),
            preferred_element_type=jnp.float32)
        carry = carry + jnp.sum(hb, axis=0, keepdims=True)

    counts = carry.astype(jnp.int32)
    pc = ((counts + (BLK - 1)) // BLK) * BLK
    pc_f = pc.astype(jnp.float32)
    ee = lax.broadcasted_iota(jnp.int32, (E, E), 0)
    ff = lax.broadcasted_iota(jnp.int32, (E, E), 1)
    le = (ff < ee).astype(jnp.float32)
    poff_f = lax.dot_general(pc_f, le, (((1,), (1,)), ((), ())),
                             preferred_element_type=jnp.float32)
    poff = poff_f.astype(jnp.int32)
    ends = poff + pc
    total = jnp.sum(pc, axis=1, keepdims=True)

    poss = lax.broadcasted_iota(jnp.int32, (NBLK, E), 0) * BLK
    eob = jnp.sum((jnp.broadcast_to(ends, (NBLK, E)) <= poss).astype(jnp.int32),
                  axis=1, keepdims=True)
    eob_ref[...] = jnp.minimum(eob, E - 1)
    act_ref[...] = (poss[:, 0:1] < total).astype(jnp.int32)

    for b in range(T // BT):
        sl = slice(b * BT, (b + 1) * BT)
        sb = scan_ref[sl, :]
        h1b, h2b = h1[sl, :], h2[sl, :]
        base0 = jnp.sum(h1b * poff_f, axis=1, keepdims=True)
        base1 = jnp.sum(h2b * poff_f, axis=1, keepdims=True)
        r0 = jnp.sum(h1b * sb, axis=1, keepdims=True)
        r1 = jnp.sum(h2b * sb, axis=1, keepdims=True)
        slot_ref[b * BT:(b + 1) * BT, :] = (base0 + r0).astype(jnp.int32)
        slot_ref[T + b * BT:T + (b + 1) * BT, :] = (base1 + r1).astype(jnp.int32)


def _run_router(x2d, rw, rb2d):
    out_shapes = (
        jax.ShapeDtypeStruct((2 * T, 1), jnp.int32),
        jax.ShapeDtypeStruct((2 * T, E), jnp.float32),
        jax.ShapeDtypeStruct((NBLK, 1), jnp.int32),
        jax.ShapeDtypeStruct((NBLK, 1), jnp.int32),
    )
    return pl.pallas_call(
        _router_body,
        out_shape=out_shapes,
        scratch_shapes=[pltpu.VMEM((T, E), jnp.float32)],
    )(x2d, rw, rb2d)


@functools.cache
def _sc_mesh():
    return plsc.VectorSubcoreMesh(core_axis_name="c", subcore_axis_name="s",
                                  num_cores=NC, num_subcores=NS)


_PAIRS_PER_W = 2 * T // NW


_HALF = _PAIRS_PER_W // 2


def _dispatch_body(x_hbm, slot_hbm, xs_hbm, slot_v0, slot_v1, rows_v0,
                   rows_v1, rsem, wsem):
    wid = lax.axis_index("s") * NC + lax.axis_index("c")
    base = wid * _PAIRS_PER_W
    tok_base = base % T
    pltpu.sync_copy(slot_hbm.at[pl.ds(base, _HALF)], slot_v0)
    pltpu.sync_copy(slot_hbm.at[pl.ds(base + _HALF, _HALF)], slot_v1)
    r0 = pltpu.async_copy(x_hbm.at[pl.ds(tok_base, _HALF)], rows_v0, rsem)
    r1 = pltpu.async_copy(x_hbm.at[pl.ds(tok_base + _HALF, _HALF)], rows_v1,
                          rsem)
    r0.wait()
    w0 = pltpu.async_copy(rows_v0, xs_hbm.at[slot_v0], wsem)
    r1.wait()
    w1 = pltpu.async_copy(rows_v1, xs_hbm.at[slot_v1], wsem)
    w0.wait()
    w1.wait()


def _run_dispatch(x2d, slot_cat):
    k = pl.kernel(
        _dispatch_body,
        out_type=jax.ShapeDtypeStruct((NSLOTS, H), jnp.float32),
        mesh=_sc_mesh(),
        scratch_types=[
            pltpu.VMEM((_HALF,), jnp.int32),
            pltpu.VMEM((_HALF,), jnp.int32),
            pltpu.VMEM((_HALF, H), jnp.float32),
            pltpu.VMEM((_HALF, H), jnp.float32),
            pltpu.SemaphoreType.DMA,
            pltpu.SemaphoreType.DMA,
        ],
    )
    return k(x2d, slot_cat)


def _expert_body(eob_ref, act_ref, x_ref, gw_ref, gb_ref, uw_ref, ub_ref,
                 dw_ref, db_ref, y_ref):
    b = pl.program_id(0)

    @pl.when(act_ref[b] == 1)
    def _():
        x = x_ref[...]
        g = lax.dot_general(x, gw_ref[0], (((1,), (1,)), ((), ())),
                            preferred_element_type=jnp.float32) + gb_ref[0]
        u = lax.dot_general(x, uw_ref[0], (((1,), (1,)), ((), ())),
                            preferred_element_type=jnp.float32) + ub_ref[0]
        a = g * jax.nn.sigmoid(g) * u
        y_ref[...] = lax.dot_general(
            a, dw_ref[0], (((1,), (1,)), ((), ())),
            preferred_element_type=jnp.float32) + db_ref[0]


def _run_experts(xs, gate_w, gate_b, up_w, up_b, down_w, down_b, eob, act):
    grid_spec = pltpu.PrefetchScalarGridSpec(
        num_scalar_prefetch=2,
        grid=(NBLK,),
        in_specs=[
            pl.BlockSpec((BLK, H), lambda b, eob, act: (act[b] * b, 0)),
            pl.BlockSpec((1, F, H), lambda b, eob, act: (eob[b], 0, 0)),
            pl.BlockSpec((1, 1, F), lambda b, eob, act: (eob[b], 0, 0)),
            pl.BlockSpec((1, F, H), lambda b, eob, act: (eob[b], 0, 0)),
            pl.BlockSpec((1, 1, F), lambda b, eob, act: (eob[b], 0, 0)),
            pl.BlockSpec((1, H, F), lambda b, eob, act: (eob[b], 0, 0)),
            pl.BlockSpec((1, 1, H), lambda b, eob, act: (eob[b], 0, 0)),
        ],
        out_specs=pl.BlockSpec((BLK, H), lambda b, eob, act: (b, 0)),
    )
    return pl.pallas_call(
        _expert_body,
        grid_spec=grid_spec,
        out_shape=jax.ShapeDtypeStruct((NSLOTS, H), jnp.float32),
        compiler_params=pltpu.CompilerParams(
            vmem_limit_bytes=128 * 1024 * 1024),
    )(eob, act, xs, gate_w, gate_b.reshape(E, 1, F), up_w,
      up_b.reshape(E, 1, F), down_w, down_b.reshape(E, 1, H))


_TOK_PER_W = T // NW
_CHUNK = 32
_NVEC = H // 16


def _combine_body(ys_hbm, slot_hbm, w_hbm, out_hbm,
                  s0_v, s1_v, a_v, b_v, w0_v, w1_v, gsem, osem):
    wid = lax.axis_index("s") * NC + lax.axis_index("c")
    nchunk = _TOK_PER_W // _CHUNK

    def fetch(c):
        base = wid * _TOK_PER_W + c * _CHUNK
        r = c % 2
        pltpu.sync_copy(slot_hbm.at[pl.ds(base, _CHUNK)], s0_v[r])
        pltpu.sync_copy(slot_hbm.at[pl.ds(T + base, _CHUNK)], s1_v[r])
        pltpu.sync_copy(w_hbm.at[pl.ds(base, _CHUNK)], w0_v[r])
        pltpu.sync_copy(w_hbm.at[pl.ds(T + base, _CHUNK)], w1_v[r])
        return (pltpu.async_copy(ys_hbm.at[s0_v[r]], a_v[r], gsem),
                pltpu.async_copy(ys_hbm.at[s1_v[r]], b_v[r], gsem))

    pend = fetch(0)
    wb = None
    for c in range(nchunk):
        r = c % 2
        if c + 1 < nchunk:
            nxt = fetch(c + 1)
        pend[0].wait()
        pend[1].wait()
        if c + 1 < nchunk:
            pend = nxt

        @plsc.parallel_loop(0, _CHUNK, unroll=2)
        def row(i):
            wa = w0_v[r][i]
            wb_ = w1_v[r][i]
            for k in range(_NVEC):
                cs = pl.ds(k * 16, 16)
                a_v[r][i, cs] = wa * a_v[r][i, cs] + wb_ * b_v[r][i, cs]
        if wb is not None:
            wb.wait()
        base = wid * _TOK_PER_W + c * _CHUNK
        wb = pltpu.async_copy(a_v[r], out_hbm.at[pl.ds(base, _CHUNK)], osem)
    wb.wait()


def _run_combine(ys, slot_cat, wcat):
    k = pl.kernel(
        _combine_body,
        out_type=jax.ShapeDtypeStruct((T, H), jnp.float32),
        mesh=_sc_mesh(),
        scratch_types=[
            [pltpu.VMEM((_CHUNK,), jnp.int32)] * 2,
            [pltpu.VMEM((_CHUNK,), jnp.int32)] * 2,
            [pltpu.VMEM((_CHUNK, H), jnp.float32)] * 2,
            [pltpu.VMEM((_CHUNK, H), jnp.float32)] * 2,
            [pltpu.VMEM((_CHUNK, E), jnp.float32)] * 2,
            [pltpu.VMEM((_CHUNK, E), jnp.float32)] * 2,
            pltpu.SemaphoreType.DMA,
            pltpu.SemaphoreType.DMA,
        ],
    )
    return k(ys, slot_cat, wcat)


@jax.jit
def kernel(hidden_states, router_w, router_b, gate_w, gate_b, up_w, up_b,
           down_w, down_b):
    x2d = hidden_states.reshape(T, H)
    slot, wcat, eob, act = _run_router(x2d, router_w, router_b.reshape(1, E))
    slot_cat = slot.reshape(2 * T)
    xs = _run_dispatch(x2d, slot_cat)
    ys = _run_experts(xs, gate_w, gate_b, up_w, up_b, down_w, down_b,
                      eob.reshape(NBLK), act.reshape(NBLK))
    out = _run_combine(ys, slot_cat, wcat)
    return out.reshape(hidden_states.shape)


# speedup vs baseline: 1.0398x; 1.0398x over previous
import functools

import jax
import jax.numpy as jnp
from jax import lax
from jax.experimental import pallas as pl
from jax.experimental.pallas import tpu as pltpu
from jax.experimental.pallas import tpu_sc as plsc

T = 2048
H = 768
F = 1024
E = 16
BLK = 288
BT = 256
NBLK = -(-T * 2 // BLK) + E
NSLOTS = NBLK * BLK

NC, NS = 2, 16
NW = NC * NS


def _router_body(x_ref, rw_ref, rb_ref,
                 slot_ref, w_ref, eob_ref, act_ref,
                 scan_ref):
    x = x_ref[...]
    logits = lax.dot_general(
        x, rw_ref[...], (((1,), (1,)), ((), ())),
        preferred_element_type=jnp.float32) + rb_ref[...]

    ids = lax.broadcasted_iota(jnp.int32, (T, E), 1)
    m1 = jnp.max(logits, axis=1, keepdims=True)
    i1 = jnp.min(jnp.where(logits == m1, ids, E), axis=1, keepdims=True)
    masked = jnp.where(ids == i1, -jnp.inf, logits)
    m2 = jnp.max(masked, axis=1, keepdims=True)
    i2 = jnp.min(jnp.where(masked == m2, ids, E), axis=1, keepdims=True)

    e2 = jnp.exp(m2 - m1)
    p1 = 1.0 / (1.0 + e2)
    p2 = 1.0 - p1
    w_ref[0:T, :] = jnp.broadcast_to(p1, (T, E))
    w_ref[T:2 * T, :] = jnp.broadcast_to(p2, (T, E))

    h1 = (ids == i1).astype(jnp.float32)
    h2 = (ids == i2).astype(jnp.float32)
    hsum = h1 + h2

    rr = lax.broadcasted_iota(jnp.int32, (BT, BT), 0)
    cc = lax.broadcasted_iota(jnp.int32, (BT, BT), 1)
    tril = (rr > cc).astype(jnp.float32)
    carry = jnp.zeros((1, E), jnp.float32)
    for b in range(T // BT):
        hb = hsum[b * BT:(b + 1) * BT, :]
        scan_ref[b * BT:(b + 1) * BT, :] = carry + lax.dot_general(
            tril, hb, (((1,), (0,)), ((), ())),
            preferred_element_type=jnp.float32)
        carry = carry + jnp.sum(hb, axis=0, keepdims=True)

    counts = carry.astype(jnp.int32)
    pc = ((counts + (BLK - 1)) // BLK) * BLK
    pc_f = pc.astype(jnp.float32)
    ee = lax.broadcasted_iota(jnp.int32, (E, E), 0)
    ff = lax.broadcasted_iota(jnp.int32, (E, E), 1)
    le = (ff < ee).astype(jnp.float32)
    poff_f = lax.dot_general(pc_f, le, (((1,), (1,)), ((), ())),
                             preferred_element_type=jnp.float32)
    poff = poff_f.astype(jnp.int32)
    ends = poff + pc
    total = jnp.sum(pc, axis=1, keepdims=True)

    poss = lax.broadcasted_iota(jnp.int32, (NBLK, E), 0) * BLK
    eob = jnp.sum((jnp.broadcast_to(ends, (NBLK, E)) <= poss).astype(jnp.int32),
                  axis=1, keepdims=True)
    eob_ref[...] = jnp.minimum(eob, E - 1)
    act_ref[...] = (poss[:, 0:1] < total).astype(jnp.int32)

    for b in range(T // BT):
        sl = slice(b * BT, (b + 1) * BT)
        sb = scan_ref[sl, :]
        h1b, h2b = h1[sl, :], h2[sl, :]
        base0 = jnp.sum(h1b * poff_f, axis=1, keepdims=True)
        base1 = jnp.sum(h2b * poff_f, axis=1, keepdims=True)
        r0 = jnp.sum(h1b * sb, axis=1, keepdims=True)
        r1 = jnp.sum(h2b * sb, axis=1, keepdims=True)
        slot_ref[b * BT:(b + 1) * BT, :] = (base0 + r0).astype(jnp.int32)
        slot_ref[T + b * BT:T + (b + 1) * BT, :] = (base1 + r1).astype(jnp.int32)


def _run_router(x2d, rw, rb2d):
    out_shapes = (
        jax.ShapeDtypeStruct((2 * T, 1), jnp.int32),
        jax.ShapeDtypeStruct((2 * T, E), jnp.float32),
        jax.ShapeDtypeStruct((NBLK, 1), jnp.int32),
        jax.ShapeDtypeStruct((NBLK, 1), jnp.int32),
    )
    return pl.pallas_call(
        _router_body,
        out_shape=out_shapes,
        scratch_shapes=[pltpu.VMEM((T, E), jnp.float32)],
    )(x2d, rw, rb2d)


@functools.cache
def _sc_mesh():
    return plsc.VectorSubcoreMesh(core_axis_name="c", subcore_axis_name="s",
                                  num_cores=NC, num_subcores=NS)


_PAIRS_PER_W = 2 * T // NW


_HALF = _PAIRS_PER_W // 2


def _dispatch_body(x_hbm, slot_hbm, xs_hbm, slot_v0, slot_v1, rows_v0,
                   rows_v1, rsem, wsem):
    wid = lax.axis_index("s") * NC + lax.axis_index("c")
    base = wid * _PAIRS_PER_W
    tok_base = base % T
    pltpu.sync_copy(slot_hbm.at[pl.ds(base, _HALF)], slot_v0)
    pltpu.sync_copy(slot_hbm.at[pl.ds(base + _HALF, _HALF)], slot_v1)
    r0 = pltpu.async_copy(x_hbm.at[pl.ds(tok_base, _HALF)], rows_v0, rsem)
    r1 = pltpu.async_copy(x_hbm.at[pl.ds(tok_base + _HALF, _HALF)], rows_v1,
                          rsem)
    r0.wait()
    w0 = pltpu.async_copy(rows_v0, xs_hbm.at[slot_v0], wsem)
    r1.wait()
    w1 = pltpu.async_copy(rows_v1, xs_hbm.at[slot_v1], wsem)
    w0.wait()
    w1.wait()


def _run_dispatch(x2d, slot_cat):
    k = pl.kernel(
        _dispatch_body,
        out_type=jax.ShapeDtypeStruct((NSLOTS, H), jnp.float32),
        mesh=_sc_mesh(),
        scratch_types=[
            pltpu.VMEM((_HALF,), jnp.int32),
            pltpu.VMEM((_HALF,), jnp.int32),
            pltpu.VMEM((_HALF, H), jnp.float32),
            pltpu.VMEM((_HALF, H), jnp.float32),
            pltpu.SemaphoreType.DMA,
            pltpu.SemaphoreType.DMA,
        ],
    )
    return k(x2d, slot_cat)


def _expert_body(eob_ref, act_ref, x_ref, gw_ref, gb_ref, uw_ref, ub_ref,
                 dw_ref, db_ref, y_ref):
    b = pl.program_id(0)

    @pl.when(act_ref[b] == 1)
    def _():
        x = x_ref[...]
        g = lax.dot_general(x, gw_ref[0], (((1,), (1,)), ((), ())),
                            preferred_element_type=jnp.float32) + gb_ref[0]
        u = lax.dot_general(x, uw_ref[0], (((1,), (1,)), ((), ())),
                            preferred_element_type=jnp.float32) + ub_ref[0]
        a = g * jax.nn.sigmoid(g) * u
        y_ref[...] = lax.dot_general(
            a, dw_ref[0], (((1,), (1,)), ((), ())),
            preferred_element_type=jnp.float32) + db_ref[0]


def _run_experts(xs, gate_w, gate_b, up_w, up_b, down_w, down_b, eob, act):
    grid_spec = pltpu.PrefetchScalarGridSpec(
        num_scalar_prefetch=2,
        grid=(NBLK,),
        in_specs=[
            pl.BlockSpec((BLK, H), lambda b, eob, act: (act[b] * b, 0)),
            pl.BlockSpec((1, F, H), lambda b, eob, act: (eob[b], 0, 0)),
            pl.BlockSpec((1, 1, F), lambda b, eob, act: (eob[b], 0, 0)),
            pl.BlockSpec((1, F, H), lambda b, eob, act: (eob[b], 0, 0)),
            pl.BlockSpec((1, 1, F), lambda b, eob, act: (eob[b], 0, 0)),
            pl.BlockSpec((1, H, F), lambda b, eob, act: (eob[b], 0, 0)),
            pl.BlockSpec((1, 1, H), lambda b, eob, act: (eob[b], 0, 0)),
        ],
        out_specs=pl.BlockSpec(
            (BLK, H),
            lambda b, eob, act: (act[b] * b + (1 - act[b]) * NBLK, 0)),
    )
    return pl.pallas_call(
        _expert_body,
        grid_spec=grid_spec,
        out_shape=jax.ShapeDtypeStruct((NSLOTS + BLK, H), jnp.float32),
        compiler_params=pltpu.CompilerParams(
            vmem_limit_bytes=128 * 1024 * 1024),
    )(eob, act, xs, gate_w, gate_b.reshape(E, 1, F), up_w,
      up_b.reshape(E, 1, F), down_w, down_b.reshape(E, 1, H))


_TOK_PER_W = T // NW
_CHUNK = 32
_NVEC = H // 16


def _combine_body(ys_hbm, slot_hbm, w_hbm, out_hbm,
                  s0_v, s1_v, a_v, b_v, w0_v, w1_v, gsem, osem):
    wid = lax.axis_index("s") * NC + lax.axis_index("c")
    nchunk = _TOK_PER_W // _CHUNK

    def fetch(c):
        base = wid * _TOK_PER_W + c * _CHUNK
        r = c % 2
        pltpu.sync_copy(slot_hbm.at[pl.ds(base, _CHUNK)], s0_v[r])
        pltpu.sync_copy(slot_hbm.at[pl.ds(T + base, _CHUNK)], s1_v[r])
        pltpu.sync_copy(w_hbm.at[pl.ds(base, _CHUNK)], w0_v[r])
        pltpu.sync_copy(w_hbm.at[pl.ds(T + base, _CHUNK)], w1_v[r])
        return (pltpu.async_copy(ys_hbm.at[s0_v[r]], a_v[r], gsem),
                pltpu.async_copy(ys_hbm.at[s1_v[r]], b_v[r], gsem))

    pend = fetch(0)
    wb = None
    for c in range(nchunk):
        r = c % 2
        if c + 1 < nchunk:
            nxt = fetch(c + 1)
        pend[0].wait()
        pend[1].wait()
        if c + 1 < nchunk:
            pend = nxt

        def row(i, _):
            wa = w0_v[r][i]
            wb_ = w1_v[r][i]
            for k in range(_NVEC):
                cs = pl.ds(k * 16, 16)
                a_v[r][i, cs] = wa * a_v[r][i, cs] + wb_ * b_v[r][i, cs]
            return 0

        lax.fori_loop(0, _CHUNK, row, 0)
        if wb is not None:
            wb.wait()
        base = wid * _TOK_PER_W + c * _CHUNK
        wb = pltpu.async_copy(a_v[r], out_hbm.at[pl.ds(base, _CHUNK)], osem)
    wb.wait()


def _run_combine(ys, slot_cat, wcat):
    k = pl.kernel(
        _combine_body,
        out_type=jax.ShapeDtypeStruct((T, H), jnp.float32),
        mesh=_sc_mesh(),
        scratch_types=[
            [pltpu.VMEM((_CHUNK,), jnp.int32)] * 2,
            [pltpu.VMEM((_CHUNK,), jnp.int32)] * 2,
            [pltpu.VMEM((_CHUNK, H), jnp.float32)] * 2,
            [pltpu.VMEM((_CHUNK, H), jnp.float32)] * 2,
            [pltpu.VMEM((_CHUNK, E), jnp.float32)] * 2,
            [pltpu.VMEM((_CHUNK, E), jnp.float32)] * 2,
            pltpu.SemaphoreType.DMA,
            pltpu.SemaphoreType.DMA,
        ],
    )
    return k(ys, slot_cat, wcat)


@jax.jit
def kernel(hidden_states, router_w, router_b, gate_w, gate_b, up_w, up_b,
           down_w, down_b):
    x2d = hidden_states.reshape(T, H)
    slot, wcat, eob, act = _run_router(x2d, router_w, router_b.reshape(1, E))
    slot_cat = slot.reshape(2 * T)
    xs = _run_dispatch(x2d, slot_cat)
    ys = _run_experts(xs, gate_w, gate_b, up_w, up_b, down_w, down_b,
                      eob.reshape(NBLK), act.reshape(NBLK))
    out = _run_combine(ys, slot_cat, wcat)
    return out.reshape(hidden_states.shape)
